# Initial kernel scaffold; baseline (speedup 1.0000x reference)
import functools

import jax
import jax.numpy as jnp
from jax import lax
from jax.experimental import pallas as pl
from jax.experimental.pallas import tpu as pltpu
from jax.experimental.pallas import tpu_sc as plsc

N_NODES = 10000
N_EDGES = 320000
D = 128

NC = 2
NS = 16
NW = NC * NS

CHUNK = 128
NCHUNK = 80
EPT = NCHUNK * CHUNK
E_PAD = EPT * NW
SINK = N_NODES
AGG_ROWS = N_NODES + 16
RPT = AGG_ROWS // NS
OPT = N_NODES // NS

_mesh = plsc.VectorSubcoreMesh(core_axis_name="c", subcore_axis_name="s")


@functools.partial(
    pl.kernel,
    mesh=_mesh,
    out_type=jax.ShapeDtypeStruct((NC, N_NODES, D), jnp.float32),
    scratch_types=[
        pltpu.VMEM((NCHUNK, CHUNK), jnp.int32),
        pltpu.VMEM((NCHUNK, CHUNK), jnp.int32),
        pltpu.VMEM((CHUNK, D), jnp.float32),
        pltpu.VMEM_SHARED((AGG_ROWS, D), jnp.float32),
        pltpu.SemaphoreType.DMA,
    ],
)
def _sc_agg(x_hbm, src_hbm, dst_hbm, zeros_hbm, out_hbm,
            src_v, dst_v, rows_v, agg_s, sem):
    c = lax.axis_index("c")
    s = lax.axis_index("s")
    wid = c * NS + s

    pltpu.sync_copy(zeros_hbm.at[pl.ds(s * RPT, RPT)],
                    agg_s.at[pl.ds(s * RPT, RPT)])
    pltpu.sync_copy(src_hbm.at[wid], src_v)
    pltpu.sync_copy(dst_hbm.at[wid], dst_v)
    plsc.subcore_barrier()

    def chunk_body(j, _):
        pltpu.async_copy(x_hbm.at[src_v.at[j]], rows_v, sem).wait()
        pltpu.sync_copy(rows_v, agg_s.at[dst_v.at[j]], add=True)
        return 0

    lax.fori_loop(0, NCHUNK, chunk_body, 0)
    plsc.subcore_barrier()

    pltpu.sync_copy(agg_s.at[pl.ds(s * OPT, OPT)],
                    out_hbm.at[c, pl.ds(s * OPT, OPT)])


BLK = 1000


def _tc_body(x_ref, a0_ref, a1_ref, w1_ref, b1_ref, w2_ref, b2_ref,
             wl_ref, bl_ref, o_ref):
    xb = x_ref[...]
    h = xb + a0_ref[...] + a1_ref[...]
    h = jnp.dot(h, w1_ref[...], preferred_element_type=jnp.float32) + b1_ref[...]
    h = jnp.maximum(h, 0.0)
    h = jnp.dot(h, w2_ref[...], preferred_element_type=jnp.float32) + b2_ref[...]
    res = jnp.dot(xb, wl_ref[...], preferred_element_type=jnp.float32) + bl_ref[...]
    o_ref[...] = res + h


def _tc_mlp(x, a0, a1, W1, b1, W2, b2, Wl, bl):
    grid = (N_NODES // BLK,)
    row_spec = pl.BlockSpec((BLK, D), lambda i: (i, 0))
    w_spec = pl.BlockSpec((D, D), lambda i: (0, 0))
    b_spec = pl.BlockSpec((1, D), lambda i: (0, 0))
    return pl.pallas_call(
        _tc_body,
        grid=grid,
        in_specs=[row_spec, row_spec, row_spec,
                  w_spec, b_spec, w_spec, b_spec, w_spec, b_spec],
        out_specs=row_spec,
        out_shape=jax.ShapeDtypeStruct((N_NODES, D), jnp.float32),
    )(x, a0, a1, W1, b1, W2, b2, Wl, bl)


@jax.jit
def kernel(x, edge_index, W1, b1, W2, b2, Wl, bl):
    ei = edge_index.astype(jnp.int32)
    pad = E_PAD - N_EDGES
    src = jnp.concatenate([ei[0], jnp.zeros((pad,), jnp.int32)])
    dst = jnp.concatenate([ei[1], jnp.full((pad,), SINK, jnp.int32)])
    src = src.reshape(NW, NCHUNK, CHUNK)
    dst = dst.reshape(NW, NCHUNK, CHUNK)
    zeros = jnp.zeros((AGG_ROWS, D), jnp.float32)
    agg = _sc_agg(x, src, dst, zeros)
    return _tc_mlp(x, agg[0], agg[1], W1,
                   b1.reshape(1, D), W2, b2.reshape(1, D),
                   Wl, bl.reshape(1, D))


# speedup vs baseline: 3.3241x; 3.3241x over previous
import functools

import jax
import jax.numpy as jnp
from jax import lax
from jax.experimental import pallas as pl
from jax.experimental.pallas import tpu as pltpu
from jax.experimental.pallas import tpu_sc as plsc

N_NODES = 10000
N_EDGES = 320000
D = 128

NC = 2
NS = 16
NW = NC * NS

CHUNK = 128
NCHUNK = 80
EPT = NCHUNK * CHUNK
E_PAD = EPT * NW
SINK = N_NODES
AGG_ROWS = 10112
RPT = AGG_ROWS // NS
OPT = 632
OPT_LAST = N_NODES - 15 * OPT

_mesh = plsc.VectorSubcoreMesh(core_axis_name="c", subcore_axis_name="s")


@functools.partial(
    pl.kernel,
    mesh=_mesh,
    out_type=jax.ShapeDtypeStruct((NC, N_NODES, D), jnp.float32),
    scratch_types=[
        pltpu.VMEM((NCHUNK, CHUNK), jnp.int32),
        pltpu.VMEM((NCHUNK, CHUNK), jnp.int32),
        pltpu.VMEM((CHUNK, D), jnp.float32),
        pltpu.VMEM_SHARED((AGG_ROWS, D), jnp.float32),
        pltpu.SemaphoreType.DMA,
    ],
)
def _sc_agg(x_hbm, src_hbm, dst_hbm, zeros_hbm, out_hbm,
            src_v, dst_v, rows_v, agg_s, sem):
    c = lax.axis_index("c")
    s = lax.axis_index("s")
    wid = c * NS + s

    pltpu.sync_copy(zeros_hbm.at[pl.ds(s * RPT, RPT)],
                    agg_s.at[pl.ds(s * RPT, RPT)])
    pltpu.sync_copy(src_hbm.at[wid], src_v)
    pltpu.sync_copy(dst_hbm.at[wid], dst_v)
    plsc.subcore_barrier()

    def chunk_body(j, _):
        pltpu.async_copy(x_hbm.at[src_v.at[j]], rows_v, sem).wait()
        pltpu.sync_copy(rows_v, agg_s.at[dst_v.at[j]], add=True)
        return 0

    lax.fori_loop(0, NCHUNK, chunk_body, 0)
    plsc.subcore_barrier()

    @pl.when(s < NS - 1)
    def _():
        pltpu.sync_copy(agg_s.at[pl.ds(s * OPT, OPT)],
                        out_hbm.at[c, pl.ds(s * OPT, OPT)])

    @pl.when(s == NS - 1)
    def _():
        pltpu.sync_copy(agg_s.at[pl.ds(15 * OPT, OPT_LAST)],
                        out_hbm.at[c, pl.ds(15 * OPT, OPT_LAST)])


BLK = 1000


def _tc_body(x_ref, a0_ref, a1_ref, w1_ref, b1_ref, w2_ref, b2_ref,
             wl_ref, bl_ref, o_ref):
    xb = x_ref[...]
    h = xb + a0_ref[...] + a1_ref[...]
    h = jnp.dot(h, w1_ref[...], preferred_element_type=jnp.float32) + b1_ref[...]
    h = jnp.maximum(h, 0.0)
    h = jnp.dot(h, w2_ref[...], preferred_element_type=jnp.float32) + b2_ref[...]
    res = jnp.dot(xb, wl_ref[...], preferred_element_type=jnp.float32) + bl_ref[...]
    o_ref[...] = res + h


def _tc_mlp(x, a0, a1, W1, b1, W2, b2, Wl, bl):
    grid = (N_NODES // BLK,)
    row_spec = pl.BlockSpec((BLK, D), lambda i: (i, 0))
    w_spec = pl.BlockSpec((D, D), lambda i: (0, 0))
    b_spec = pl.BlockSpec((1, D), lambda i: (0, 0))
    return pl.pallas_call(
        _tc_body,
        grid=grid,
        in_specs=[row_spec, row_spec, row_spec,
                  w_spec, b_spec, w_spec, b_spec, w_spec, b_spec],
        out_specs=row_spec,
        out_shape=jax.ShapeDtypeStruct((N_NODES, D), jnp.float32),
    )(x, a0, a1, W1, b1, W2, b2, Wl, bl)


@jax.jit
def kernel(x, edge_index, W1, b1, W2, b2, Wl, bl):
    ei = edge_index.astype(jnp.int32)
    pad = E_PAD - N_EDGES
    src = jnp.concatenate([ei[0], jnp.zeros((pad,), jnp.int32)])
    dst = jnp.concatenate([ei[1], jnp.full((pad,), SINK, jnp.int32)])
    src = src.reshape(NW, NCHUNK, CHUNK)
    dst = dst.reshape(NW, NCHUNK, CHUNK)
    zeros = jnp.zeros((AGG_ROWS, D), jnp.float32)
    agg = _sc_agg(x, src, dst, zeros)
    return _tc_mlp(x, agg[0], agg[1], W1,
                   b1.reshape(1, D), W2, b2.reshape(1, D),
                   Wl, bl.reshape(1, D))


# speedup vs baseline: 6.5866x; 1.9815x over previous
import functools

import jax
import jax.numpy as jnp
from jax import lax
from jax.experimental import pallas as pl
from jax.experimental.pallas import tpu as pltpu
from jax.experimental.pallas import tpu_sc as plsc

N_NODES = 10000
N_EDGES = 320000
D = 128

NC = 2
NS = 16
NW = NC * NS

CHUNK = 96
NBUF = 3
NGROUP = 35
NCHUNK = NGROUP * NBUF
EPT = NCHUNK * CHUNK
E_PAD = EPT * NW
SINK = N_NODES
AGG_ROWS = 10112
RPT = AGG_ROWS // NS
OPT = 632
OPT_LAST = N_NODES - 15 * OPT

_mesh = plsc.VectorSubcoreMesh(core_axis_name="c", subcore_axis_name="s")


@functools.partial(
    pl.kernel,
    mesh=_mesh,
    out_type=jax.ShapeDtypeStruct((NC, N_NODES, D), jnp.float32),
    scratch_types=[
        pltpu.VMEM((2, NBUF, CHUNK), jnp.int32),
        pltpu.VMEM((2, NBUF, CHUNK), jnp.int32),
        pltpu.VMEM((NBUF, CHUNK, D), jnp.float32),
        pltpu.VMEM_SHARED((AGG_ROWS, D), jnp.float32),
        pltpu.SemaphoreType.DMA((NBUF,)),
        pltpu.SemaphoreType.DMA((NBUF,)),
        pltpu.SemaphoreType.DMA((2,)),
        pltpu.SemaphoreType.DMA((2,)),
    ],
)
def _sc_agg(x_hbm, src_hbm, dst_hbm, zeros_hbm, out_hbm,
            sidx, didx, rows_v, agg_s, gsem, ssem, fsem_s, fsem_d):
    c = lax.axis_index("c")
    s = lax.axis_index("s")
    wid = c * NS + s

    pltpu.sync_copy(zeros_hbm.at[pl.ds(s * RPT, RPT)],
                    agg_s.at[pl.ds(s * RPT, RPT)])
    plsc.subcore_barrier()

    def gather(p, b):
        pltpu.async_copy(x_hbm.at[sidx.at[p, b]], rows_v.at[b], gsem.at[b])

    def gather_wait(p, b):
        pltpu.make_async_copy(x_hbm.at[sidx.at[p, b]], rows_v.at[b],
                              gsem.at[b]).wait()

    def scatter_start(p, b):
        pltpu.async_copy(rows_v.at[b], agg_s.at[didx.at[p, b]], ssem.at[b],
                         add=True)

    def scatter_wait(p, b):
        pltpu.make_async_copy(rows_v.at[b], agg_s.at[didx.at[p, b]],
                              ssem.at[b]).wait()

    pltpu.sync_copy(src_hbm.at[wid, 0], sidx.at[0])
    pltpu.sync_copy(dst_hbm.at[wid, 0], didx.at[0])
    for b in range(NBUF):
        gather(0, b)

    def group_body(g, _):
        p = lax.rem(g, 2)
        q = 1 - p
        have_next = g + 1 < NGROUP

        @pl.when(have_next)
        def _():
            pltpu.async_copy(src_hbm.at[wid, g + 1], sidx.at[q], fsem_s.at[q])
            pltpu.async_copy(dst_hbm.at[wid, g + 1], didx.at[q], fsem_d.at[q])

        for b in range(NBUF):
            gather_wait(p, b)
            scatter_start(p, b)

        @pl.when(have_next)
        def _():
            pltpu.make_async_copy(src_hbm.at[wid, g + 1], sidx.at[q],
                                  fsem_s.at[q]).wait()
            pltpu.make_async_copy(dst_hbm.at[wid, g + 1], didx.at[q],
                                  fsem_d.at[q]).wait()

        for b in range(NBUF):
            scatter_wait(p, b)

            @pl.when(have_next)
            def _():
                gather(q, b)

        return 0

    lax.fori_loop(0, NGROUP, group_body, 0)
    plsc.subcore_barrier()

    @pl.when(s < NS - 1)
    def _():
        pltpu.sync_copy(agg_s.at[pl.ds(s * OPT, OPT)],
                        out_hbm.at[c, pl.ds(s * OPT, OPT)])

    @pl.when(s == NS - 1)
    def _():
        pltpu.sync_copy(agg_s.at[pl.ds(15 * OPT, OPT_LAST)],
                        out_hbm.at[c, pl.ds(15 * OPT, OPT_LAST)])


BLK = 1000


def _tc_body(x_ref, a0_ref, a1_ref, w1_ref, b1_ref, w2_ref, b2_ref,
             wl_ref, bl_ref, o_ref):
    xb = x_ref[...]
    h = xb + a0_ref[...] + a1_ref[...]
    h = jnp.dot(h, w1_ref[...], preferred_element_type=jnp.float32) + b1_ref[...]
    h = jnp.maximum(h, 0.0)
    h = jnp.dot(h, w2_ref[...], preferred_element_type=jnp.float32) + b2_ref[...]
    res = jnp.dot(xb, wl_ref[...], preferred_element_type=jnp.float32) + bl_ref[...]
    o_ref[...] = res + h


def _tc_mlp(x, a0, a1, W1, b1, W2, b2, Wl, bl):
    grid = (N_NODES // BLK,)
    row_spec = pl.BlockSpec((BLK, D), lambda i: (i, 0))
    w_spec = pl.BlockSpec((D, D), lambda i: (0, 0))
    b_spec = pl.BlockSpec((1, D), lambda i: (0, 0))
    return pl.pallas_call(
        _tc_body,
        grid=grid,
        in_specs=[row_spec, row_spec, row_spec,
                  w_spec, b_spec, w_spec, b_spec, w_spec, b_spec],
        out_specs=row_spec,
        out_shape=jax.ShapeDtypeStruct((N_NODES, D), jnp.float32),
    )(x, a0, a1, W1, b1, W2, b2, Wl, bl)


@jax.jit
def kernel(x, edge_index, W1, b1, W2, b2, Wl, bl):
    ei = edge_index.astype(jnp.int32)
    pad = E_PAD - N_EDGES
    src = jnp.concatenate([ei[0], jnp.zeros((pad,), jnp.int32)])
    dst = jnp.concatenate([ei[1], jnp.full((pad,), SINK, jnp.int32)])
    src = src.reshape(NW, NGROUP, NBUF, CHUNK)
    dst = dst.reshape(NW, NGROUP, NBUF, CHUNK)
    zeros = jnp.zeros((AGG_ROWS, D), jnp.float32)
    agg = _sc_agg(x, src, dst, zeros)
    return _tc_mlp(x, agg[0], agg[1], W1,
                   b1.reshape(1, D), W2, b2.reshape(1, D),
                   Wl, bl.reshape(1, D))
